# baseline (device time: 15739 ns/iter reference)
import jax
import jax.numpy as jnp
from jax import lax
from jax.experimental import pallas as pl
from jax.experimental.pallas import tpu as pltpu

N_DEV = 4
N_COL_CHUNKS = 2


def kernel(x):
    m_per, n = x.shape
    n_chunk = n // N_COL_CHUNKS

    def body(x_ref, out_ref, comm_ref, send_sems, recv_sems):
        step = pl.program_id(0)
        my_pos = lax.axis_index("i")

        @pl.when(step == 0)
        def _():
            barrier_sem = pltpu.get_barrier_semaphore()
            for j in range(1, N_DEV):
                pl.semaphore_signal(
                    barrier_sem,
                    inc=1,
                    device_id=((my_pos + j) % N_DEV,),
                    device_id_type=pl.DeviceIdType.MESH,
                )
            pl.semaphore_wait(barrier_sem, N_DEV - 1)

        def exchange(c):
            comm_ref[c, 0, :, :] = jnp.sum(x_ref[:, :], axis=0, keepdims=True)
            for j in range(1, N_DEV):
                s = N_DEV - j
                rdma = pltpu.make_async_remote_copy(
                    src_ref=comm_ref.at[c, 0],
                    dst_ref=comm_ref.at[c, s],
                    send_sem=send_sems.at[c, j - 1],
                    recv_sem=recv_sems.at[c, s - 1],
                    device_id=((my_pos + j) % N_DEV,),
                    device_id_type=pl.DeviceIdType.MESH,
                )
                rdma.start()

        for c in range(N_COL_CHUNKS):
            @pl.when(step == c)
            def _(c=c):
                exchange(c)

        @pl.when(step == N_COL_CHUNKS - 1)
        def _():
            for c in range(N_COL_CHUNKS):
                for j in range(1, N_DEV):
                    s = N_DEV - j
                    rdma = pltpu.make_async_remote_copy(
                        src_ref=comm_ref.at[c, 0],
                        dst_ref=comm_ref.at[c, s],
                        send_sem=send_sems.at[c, j - 1],
                        recv_sem=recv_sems.at[c, s - 1],
                        device_id=((my_pos + j) % N_DEV,),
                        device_id_type=pl.DeviceIdType.MESH,
                    )
                    rdma.wait()
            scale = 1.0 / (N_DEV * m_per)
            for c in range(N_COL_CHUNKS):
                total = (
                    comm_ref[c, 0, :, :]
                    + comm_ref[c, 1, :, :]
                    + comm_ref[c, 2, :, :]
                    + comm_ref[c, 3, :, :]
                )
                out_ref[:, pl.ds(c * n_chunk, n_chunk)] = total * scale

    return pl.pallas_call(
        body,
        grid=(N_COL_CHUNKS,),
        out_shape=jax.ShapeDtypeStruct((1, n), jnp.float32),
        in_specs=[
            pl.BlockSpec((m_per, n_chunk), lambda i: (0, i), memory_space=pltpu.VMEM)
        ],
        out_specs=pl.BlockSpec((1, n), lambda i: (0, 0), memory_space=pltpu.VMEM),
        scratch_shapes=[
            pltpu.VMEM((N_COL_CHUNKS, N_DEV, 1, n_chunk), jnp.float32),
            pltpu.SemaphoreType.DMA((N_COL_CHUNKS, N_DEV - 1)),
            pltpu.SemaphoreType.DMA((N_COL_CHUNKS, N_DEV - 1)),
        ],
        compiler_params=pltpu.CompilerParams(collective_id=0),
    )(x)


# device time: 13299 ns/iter; 1.1835x vs baseline; 1.1835x over previous
import jax
import jax.numpy as jnp
from jax import lax
from jax.experimental import pallas as pl
from jax.experimental.pallas import tpu as pltpu

N_DEV = 4
N_ROW_CHUNKS = 4


def kernel(x):
    m_per, n = x.shape
    m_chunk = m_per // N_ROW_CHUNKS

    def body(x_hbm, out_ref, x_vmem, comm_ref, copy_sems, send_sems, recv_sems):
        my_pos = lax.axis_index("i")

        copies = []
        for r in range(N_ROW_CHUNKS):
            cp = pltpu.make_async_copy(
                x_hbm.at[pl.ds(r * m_chunk, m_chunk), :],
                x_vmem.at[r],
                copy_sems.at[r],
            )
            cp.start()
            copies.append(cp)

        barrier_sem = pltpu.get_barrier_semaphore()
        for j in range(1, N_DEV):
            pl.semaphore_signal(
                barrier_sem,
                inc=1,
                device_id=((my_pos + j) % N_DEV,),
                device_id_type=pl.DeviceIdType.MESH,
            )
        pl.semaphore_wait(barrier_sem, N_DEV - 1)

        acc = None
        for r in range(N_ROW_CHUNKS):
            copies[r].wait()
            part = jnp.sum(x_vmem[r], axis=0, keepdims=True)
            acc = part if acc is None else acc + part
        comm_ref[0, :, :] = acc

        rdmas = []
        for j in range(1, N_DEV):
            s = N_DEV - j
            rdma = pltpu.make_async_remote_copy(
                src_ref=comm_ref.at[0],
                dst_ref=comm_ref.at[s],
                send_sem=send_sems.at[j - 1],
                recv_sem=recv_sems.at[s - 1],
                device_id=((my_pos + j) % N_DEV,),
                device_id_type=pl.DeviceIdType.MESH,
            )
            rdma.start()
            rdmas.append(rdma)

        for rdma in rdmas:
            rdma.wait()

        total = (
            comm_ref[0, :, :]
            + comm_ref[1, :, :]
            + comm_ref[2, :, :]
            + comm_ref[3, :, :]
        )
        out_ref[:, :] = total * (1.0 / (N_DEV * m_per))

    return pl.pallas_call(
        body,
        out_shape=jax.ShapeDtypeStruct((1, n), jnp.float32),
        in_specs=[pl.BlockSpec(memory_space=pl.ANY)],
        out_specs=pl.BlockSpec(memory_space=pltpu.VMEM),
        scratch_shapes=[
            pltpu.VMEM((N_ROW_CHUNKS, m_chunk, n), jnp.float32),
            pltpu.VMEM((N_DEV, 1, n), jnp.float32),
            pltpu.SemaphoreType.DMA((N_ROW_CHUNKS,)),
            pltpu.SemaphoreType.DMA((N_DEV - 1,)),
            pltpu.SemaphoreType.DMA((N_DEV - 1,)),
        ],
        compiler_params=pltpu.CompilerParams(collective_id=0),
    )(x)


# device time: 11665 ns/iter; 1.3492x vs baseline; 1.1401x over previous
import jax
import jax.numpy as jnp
from jax import lax
from jax.experimental import pallas as pl
from jax.experimental.pallas import tpu as pltpu

N_DEV = 4


def kernel(x):
    m_per, n = x.shape

    def body(x_ref, out_ref, comm_ref, send_sems, recv_sems):
        my_pos = lax.axis_index("i")

        barrier_sem = pltpu.get_barrier_semaphore()
        for j in range(1, N_DEV):
            pl.semaphore_signal(
                barrier_sem,
                inc=1,
                device_id=((my_pos + j) % N_DEV,),
                device_id_type=pl.DeviceIdType.MESH,
            )

        scale = 1.0 / (N_DEV * m_per)
        comm_ref[0, :, :] = jnp.sum(x_ref[:, :], axis=0, keepdims=True) * scale

        pl.semaphore_wait(barrier_sem, N_DEV - 1)

        rdmas = {}
        for j in range(1, N_DEV):
            s = N_DEV - j
            rdma = pltpu.make_async_remote_copy(
                src_ref=comm_ref.at[0],
                dst_ref=comm_ref.at[s],
                send_sem=send_sems.at[j - 1],
                recv_sem=recv_sems.at[s - 1],
                device_id=((my_pos + j) % N_DEV,),
                device_id_type=pl.DeviceIdType.MESH,
            )
            rdma.start()
            rdmas[s] = rdma

        rdmas[1].wait_recv()
        rdmas[3].wait_recv()
        partial3 = comm_ref[0, :, :] + comm_ref[1, :, :] + comm_ref[3, :, :]
        rdmas[2].wait_recv()
        out_ref[:, :] = partial3 + comm_ref[2, :, :]

        for s in (1, 2, 3):
            rdmas[s].wait_send()

    return pl.pallas_call(
        body,
        out_shape=jax.ShapeDtypeStruct((1, n), jnp.float32),
        in_specs=[pl.BlockSpec(memory_space=pltpu.VMEM)],
        out_specs=pl.BlockSpec(memory_space=pltpu.VMEM),
        scratch_shapes=[
            pltpu.VMEM((N_DEV, 1, n), jnp.float32),
            pltpu.SemaphoreType.DMA((N_DEV - 1,)),
            pltpu.SemaphoreType.DMA((N_DEV - 1,)),
        ],
        compiler_params=pltpu.CompilerParams(collective_id=0),
    )(x)
